# baseline (speedup 1.0000x reference)
import functools

import jax
import jax.numpy as jnp
from jax import lax
from jax.experimental import pallas as pl
from jax.experimental.pallas import tpu as pltpu
from jax.experimental.pallas import tpu_sc as plsc

_N = 4096
_C = 26
_DEPTH = 1000
_NC = 1
_NW = 16 * _NC
_RPW = _N // _NW
_R = 1
_NCH = _RPW // _R
_PAD = 32
_NVEC = _PAD // 16


def _sc_body(idxpad_hbm, rpat_hbm, cpat_hbm, zeros_hbm, out_hbm,
             idx_v, rpat_v, cpat_v, buf0, buf1, sem0, sem1):
    wid = lax.axis_index("s") * _NC + lax.axis_index("c")

    pltpu.sync_copy(idxpad_hbm.at[pl.ds(wid * (_NCH * _PAD), _NCH * _PAD)],
                    idx_v)
    pltpu.sync_copy(rpat_hbm, rpat_v)
    pltpu.sync_copy(cpat_hbm, cpat_v)
    pltpu.sync_copy(zeros_hbm, buf0)
    pltpu.sync_copy(zeros_hbm, buf1)

    ones = jnp.full((16,), 1.0, jnp.float32)
    zeros = jnp.zeros((16,), jnp.float32)
    row_base = wid * _RPW

    def scatter(buf, ch, val):
        for t in range(_NVEC):
            iv = idx_v[pl.ds(ch * _PAD + t * 16, 16)]
            rv = rpat_v[pl.ds(t * 16, 16)]
            cv = cpat_v[pl.ds(t * 16, 16)]
            plsc.store_scatter(buf, [rv, cv, iv], val, mask=iv >= 0)

    def body(g, carry):
        for b, (buf, sem) in enumerate(((buf0, sem0), (buf1, sem1))):
            ch = 2 * g + b

            @pl.when(g >= 1)
            def _recycle():
                pltpu.make_async_copy(
                    buf, out_hbm.at[pl.ds(row_base, _R)], sem).wait()
                scatter(buf, ch - 2, zeros)

            scatter(buf, ch, ones)
            pltpu.make_async_copy(
                buf, out_hbm.at[pl.ds(row_base + ch * _R, _R)], sem).start()
        return carry

    lax.fori_loop(0, _NCH // 2, body, 0)

    pltpu.make_async_copy(buf0, out_hbm.at[pl.ds(row_base, _R)], sem0).wait()
    pltpu.make_async_copy(buf1, out_hbm.at[pl.ds(row_base, _R)], sem1).wait()


@jax.jit
def _one_hot_sc(idx_pad_flat, rpat, cpat, zeros_chunk):
    mesh = plsc.VectorSubcoreMesh(core_axis_name="c", subcore_axis_name="s",
                                  num_cores=_NC, num_subcores=16)
    return pl.kernel(
        _sc_body,
        out_type=jax.ShapeDtypeStruct((_N, _C, _DEPTH), jnp.float32),
        mesh=mesh,
        compiler_params=pltpu.CompilerParams(needs_layout_passes=False),
        scratch_types=[
            pltpu.VMEM((_NCH * _PAD,), jnp.int32),
            pltpu.VMEM((_PAD,), jnp.int32),
            pltpu.VMEM((_PAD,), jnp.int32),
            pltpu.VMEM((_R, _C, _DEPTH), jnp.float32),
            pltpu.VMEM((_R, _C, _DEPTH), jnp.float32),
            pltpu.SemaphoreType.DMA,
            pltpu.SemaphoreType.DMA,
        ],
    )(idx_pad_flat, rpat, cpat, zeros_chunk)


def kernel(inputs):
    idx = inputs.astype(jnp.int32)
    chunks = idx.reshape(_N // _R, _R * _C)
    pad = jnp.full((_N // _R, _PAD - _R * _C), -1, jnp.int32)
    idx_pad_flat = jnp.concatenate([chunks, pad], axis=1).reshape(-1)
    slot = jnp.arange(_PAD, dtype=jnp.int32)
    valid = slot < _R * _C
    rpat = jnp.where(valid, slot // _C, 0)
    cpat = jnp.where(valid, slot % _C, 0)
    zeros_chunk = jnp.zeros((_R, _C, _DEPTH), jnp.float32)
    return _one_hot_sc(idx_pad_flat, rpat, cpat, zeros_chunk)


# speedup vs baseline: 1.2009x; 1.2009x over previous
import jax
import jax.numpy as jnp
from jax import lax
from jax.experimental import pallas as pl
from jax.experimental.pallas import tpu as pltpu
from jax.experimental.pallas import tpu_sc as plsc

_N = 4096
_C = 26
_DEPTH = 1000
_NW = 32
_RPW = _N // _NW
_R = 1
_NCH = _RPW // _R
_PAD = 32
_NVEC = _PAD // 16


def _sc_body(idxpad_hbm, rpat_hbm, cpat_hbm, zeros_hbm, out_hbm,
             idx_v, rpat_v, cpat_v, buf0, buf1, sem0, sem1):
    wid = lax.axis_index("s") * 2 + lax.axis_index("c")

    pltpu.sync_copy(idxpad_hbm.at[pl.ds(wid * (_NCH * _PAD), _NCH * _PAD)],
                    idx_v)
    pltpu.sync_copy(rpat_hbm, rpat_v)
    pltpu.sync_copy(cpat_hbm, cpat_v)
    pltpu.sync_copy(zeros_hbm, buf0)
    pltpu.sync_copy(zeros_hbm, buf1)

    ones = jnp.full((16,), 1.0, jnp.float32)
    zeros = jnp.zeros((16,), jnp.float32)
    row_base = wid * _RPW

    def scatter(buf, ch, val):
        for t in range(_NVEC):
            iv = idx_v[pl.ds(ch * _PAD + t * 16, 16)]
            rv = rpat_v[pl.ds(t * 16, 16)]
            cv = cpat_v[pl.ds(t * 16, 16)]
            plsc.store_scatter(buf, [rv, cv, iv], val, mask=iv >= 0)

    def body(g, carry):
        for b, (buf, sem) in enumerate(((buf0, sem0), (buf1, sem1))):
            ch = 2 * g + b

            @pl.when(g >= 1)
            def _recycle():
                pltpu.make_async_copy(
                    buf, out_hbm.at[pl.ds(row_base, _R)], sem).wait()
                scatter(buf, ch - 2, zeros)

            scatter(buf, ch, ones)
            pltpu.make_async_copy(
                buf, out_hbm.at[pl.ds(row_base + ch * _R, _R)], sem).start()
        return carry

    lax.fori_loop(0, _NCH // 2, body, 0)

    pltpu.make_async_copy(buf0, out_hbm.at[pl.ds(row_base, _R)], sem0).wait()
    pltpu.make_async_copy(buf1, out_hbm.at[pl.ds(row_base, _R)], sem1).wait()


@jax.jit
def _one_hot_sc(idx_pad_flat, rpat, cpat, zeros_chunk):
    mesh = plsc.VectorSubcoreMesh(core_axis_name="c", subcore_axis_name="s",
                                  num_cores=2, num_subcores=16)
    return pl.kernel(
        _sc_body,
        out_type=jax.ShapeDtypeStruct((_N, _C, _DEPTH), jnp.float32),
        mesh=mesh,
        compiler_params=pltpu.CompilerParams(needs_layout_passes=False),
        scratch_types=[
            pltpu.VMEM((_NCH * _PAD,), jnp.int32),
            pltpu.VMEM((_PAD,), jnp.int32),
            pltpu.VMEM((_PAD,), jnp.int32),
            pltpu.VMEM((_R, _C, _DEPTH), jnp.float32),
            pltpu.VMEM((_R, _C, _DEPTH), jnp.float32),
            pltpu.SemaphoreType.DMA,
            pltpu.SemaphoreType.DMA,
        ],
    )(idx_pad_flat, rpat, cpat, zeros_chunk)


def kernel(inputs):
    idx = inputs.astype(jnp.int32)
    chunks = idx.reshape(_N // _R, _R * _C)
    pad = jnp.full((_N // _R, _PAD - _R * _C), -1, jnp.int32)
    idx_pad_flat = jnp.concatenate([chunks, pad], axis=1).reshape(-1)
    slot = jnp.arange(_PAD, dtype=jnp.int32)
    valid = slot < _R * _C
    rpat = jnp.where(valid, slot // _C, 0)
    cpat = jnp.where(valid, slot % _C, 0)
    zeros_chunk = jnp.zeros((_R, _C, _DEPTH), jnp.float32)
    return _one_hot_sc(idx_pad_flat, rpat, cpat, zeros_chunk)
